# baseline (speedup 1.0000x reference)
import functools

import jax
import jax.numpy as jnp
from jax import lax
from jax.experimental import pallas as pl
from jax.experimental.pallas import tpu as pltpu
from jax.experimental.pallas import tpu_sc as plsc

N_NODES = 10000
N_EDGES = 80000
IN_CH = 32
GROWTH = 16
HIDDEN = 32
BETA = 0.2

NC = 2
NS = 16
NW = NC * NS
CHUNK = 128
EPW = 2560
NCHUNK = EPW // CHUNK
E_PAD = NW * EPW
N_PAD = 10240
ZW = 48

SELU_L = 1.0507009873554805
SELU_A = 1.6732632423543772

_mesh = plsc.VectorSubcoreMesh(core_axis_name="c", subcore_axis_name="s")
_sc_params = pltpu.CompilerParams(use_tc_tiling_on_sc=False,
                                  needs_layout_passes=False)


def _selu(x):
    return SELU_L * jnp.where(x > 0, x, SELU_A * (jnp.exp(x) - 1.0))


@functools.partial(
    pl.kernel,
    out_type=jax.ShapeDtypeStruct((6, E_PAD), jnp.float32),
    mesh=_mesh,
    compiler_params=_sc_params,
    scratch_types=[
        pltpu.VMEM((NCHUNK, CHUNK), jnp.int32),
        pltpu.VMEM((NCHUNK, CHUNK), jnp.int32),
        pltpu.VMEM((EPW, 16), jnp.float32),
        pltpu.VMEM((EPW, 16), jnp.float32),
        pltpu.VMEM((6, EPW), jnp.float32),
        pltpu.SemaphoreType.DMA,
    ],
)
def _sc_gather_pos(pos16, idx_s_hbm, idx_d_hbm, out, idx_s, idx_d,
                   rows_s, rows_d, cols, sem):
    wid = lax.axis_index("s") * NC + lax.axis_index("c")
    pltpu.sync_copy(idx_s_hbm.at[wid], idx_s)
    pltpu.sync_copy(idx_d_hbm.at[wid], idx_d)

    @pl.loop(0, NCHUNK)
    def _fire(j):
        pltpu.async_copy(pos16.at[idx_s.at[j]],
                         rows_s.at[pl.ds(j * CHUNK, CHUNK)], sem)
        pltpu.async_copy(pos16.at[idx_d.at[j]],
                         rows_d.at[pl.ds(j * CHUNK, CHUNK)], sem)

    @pl.loop(0, 2 * NCHUNK)
    def _drain(j):
        pltpu.make_async_copy(pos16.at[idx_s.at[0]],
                              rows_s.at[pl.ds(0, CHUNK)], sem).wait()

    lanes = lax.iota(jnp.int32, 16)

    @pl.loop(0, EPW // 16)
    def _tr(p):
        e_idx = lanes + p * 16
        for t in range(6):
            buf = rows_s if t < 3 else rows_d
            cidx = jnp.full((16,), t % 3, jnp.int32)
            cols[t, pl.ds(p * 16, 16)] = plsc.load_gather(buf, [e_idx, cidx])

    @pl.loop(0, 6)
    def _out(t):
        pltpu.sync_copy(cols.at[t], out.at[t].at[pl.ds(wid * EPW, EPW)])


@functools.partial(
    pl.kernel,
    out_type=jax.ShapeDtypeStruct((E_PAD, ZW), jnp.float32),
    mesh=_mesh,
    compiler_params=_sc_params,
    scratch_types=[
        pltpu.VMEM((NCHUNK, CHUNK), jnp.int32),
        pltpu.VMEM((EPW, ZW), jnp.float32),
        pltpu.SemaphoreType.DMA,
    ],
)
def _sc_gather_z(table, idx_hbm, out, idx_v, rows, sem):
    wid = lax.axis_index("s") * NC + lax.axis_index("c")
    pltpu.sync_copy(idx_hbm.at[wid], idx_v)

    @pl.loop(0, NCHUNK)
    def _fire(j):
        pltpu.async_copy(table.at[idx_v.at[j]],
                         rows.at[pl.ds(j * CHUNK, CHUNK)], sem)

    @pl.loop(0, NCHUNK)
    def _drain(j):
        pltpu.make_async_copy(table.at[idx_v.at[0]],
                              rows.at[pl.ds(0, CHUNK)], sem).wait()

    pltpu.sync_copy(rows, out.at[pl.ds(wid * EPW, EPW)])


def _make_sc_scatter(oc):
    rpt = N_PAD // NS

    @functools.partial(
        pl.kernel,
        out_type=jax.ShapeDtypeStruct((NC, N_PAD, oc), jnp.float32),
        mesh=_mesh,
        compiler_params=_sc_params,
        scratch_types=[
            pltpu.VMEM((NCHUNK, CHUNK), jnp.int32),
            pltpu.VMEM((EPW, oc), jnp.float32),
            pltpu.VMEM_SHARED((N_PAD, oc), jnp.float32),
            pltpu.SemaphoreType.DMA,
        ],
    )
    def _sc_scatter(msg_hbm, idx_hbm, zeros_hbm, out, idx_v, msg_v, acc, sem):
        cid = lax.axis_index("c")
        sid = lax.axis_index("s")
        wid = sid * NC + cid
        pltpu.sync_copy(zeros_hbm.at[pl.ds(sid * rpt, rpt)],
                        acc.at[pl.ds(sid * rpt, rpt)])
        pltpu.sync_copy(idx_hbm.at[wid], idx_v)
        pltpu.sync_copy(msg_hbm.at[pl.ds(wid * EPW, EPW)], msg_v)
        plsc.subcore_barrier()

        @pl.loop(0, NCHUNK)
        def _scat(j):
            pltpu.sync_copy(msg_v.at[pl.ds(j * CHUNK, CHUNK)],
                            acc.at[idx_v.at[j]], add=True)

        plsc.subcore_barrier()
        pltpu.sync_copy(acc.at[pl.ds(sid * rpt, rpt)],
                        out.at[cid].at[pl.ds(sid * rpt, rpt)])

    return _sc_scatter


_sc_scatter_16 = _make_sc_scatter(GROWTH)
_sc_scatter_32 = _make_sc_scatter(IN_CH)


_HI = lax.Precision.HIGHEST
_tc_params = pltpu.CompilerParams(vmem_limit_bytes=60000 * 1024)


def _mm_body(x_r, w_r, o_r):
    o_r[...] = jnp.dot(x_r[...], w_r[...], precision=_HI,
                       preferred_element_type=jnp.float32)


def _tc_matmul(x, w):
    m, k = x.shape
    _, n = w.shape
    return pl.pallas_call(
        _mm_body,
        out_shape=jax.ShapeDtypeStruct((m, n), jnp.float32),
        compiler_params=_tc_params,
    )(x, w)


_PI = 3.141592653589793


def _atan(t):
    s = jnp.sign(t)
    a = jnp.abs(t)
    inv = a > 1.0
    a1 = jnp.where(inv, 1.0 / jnp.maximum(a, 1e-30), a)
    big = a1 > 0.4142135623730951
    a2 = jnp.where(big, (a1 - 1.0) / (a1 + 1.0), a1)
    z = a2 * a2
    y = ((8.05374449538e-2 * z - 1.38776856032e-1) * z
         + 1.99777106478e-1) * z - 3.33329491539e-1
    y = y * z * a2 + a2
    y = jnp.where(big, y + 0.7853981633974483, y)
    y = jnp.where(inv, 1.5707963267948966 - y, y)
    return s * y


def _atan2(y, x):
    ax = jnp.abs(x)
    ay = jnp.abs(y)
    t = ay / jnp.where(ax > 0, ax, 1.0)
    r = _atan(t)
    r = jnp.where(x < 0, _PI - r, r)
    r = jnp.where(jnp.maximum(ax, ay) == 0.0, 0.0, r)
    return jnp.where(y < 0, -r, r)


def _feat_body(p_r, orho, oth, oph):
    rx = p_r[3] - p_r[0]
    ry = p_r[4] - p_r[1]
    rz = p_r[5] - p_r[2]
    rho = jnp.sqrt(rx * rx + ry * ry + rz * rz)
    theta = _atan2(ry, rx)
    safe = jnp.where(rho > 0, rho, 1.0)
    u = jnp.clip(rz / safe, -1.0, 1.0)
    phi = _atan2(u, jnp.sqrt(jnp.maximum(1.0 - u * u, 0.0)))
    m = rho > 0
    orho[...] = rho
    oth[...] = jnp.where(m, theta, 0.0) * (1.0 / _PI)
    oph[...] = jnp.where(m, phi, 0.0) * (1.0 / _PI)


def _tc_feat(pos_g):
    rows = E_PAD // 128
    shp = jax.ShapeDtypeStruct((rows, 128), jnp.float32)
    return pl.pallas_call(
        _feat_body,
        out_shape=(shp, shp, shp),
        compiler_params=_tc_params,
    )(pos_g)


def _make_msg_body(oc, nblk):
    def body(rho_r, th_r, ph_r, z_r, w0_r, cw_r, seg_r, o_r):
        def bf(v):
            return v.astype(jnp.bfloat16).astype(jnp.float32)
        g0 = (bf(rho_r[...]) * w0_r[0:1, :] + bf(th_r[...]) * w0_r[1:2, :]
              + bf(ph_r[...]) * w0_r[2:3, :])
        gfull = jnp.concatenate([g0] * oc, axis=1) + cw_r[...]
        s_elu = bf(_selu(gfull))
        z = z_r[...]
        zt = jnp.concatenate([z[:, :HIDDEN]] * oc, axis=1)
        msg = jnp.dot(s_elu * zt, seg_r[...], precision=_HI,
                      preferred_element_type=jnp.float32)
        o_r[...] = msg + z[:, HIDDEN:HIDDEN + 1]
    return body


def _tc_msg(rho, th, ph, zsrc, w0r, cw, seg, oc, blk=2048):
    nblk = E_PAD // blk
    k = oc * HIDDEN
    return pl.pallas_call(
        _make_msg_body(oc, nblk),
        grid=(nblk,),
        in_specs=[
            pl.BlockSpec((blk, 1), lambda i: (i, 0)),
            pl.BlockSpec((blk, 1), lambda i: (i, 0)),
            pl.BlockSpec((blk, 1), lambda i: (i, 0)),
            pl.BlockSpec((blk, ZW), lambda i: (i, 0)),
            pl.BlockSpec((3, HIDDEN), lambda i: (0, 0)),
            pl.BlockSpec((1, k), lambda i: (0, 0)),
            pl.BlockSpec((k, oc), lambda i: (0, 0)),
        ],
        out_specs=pl.BlockSpec((blk, oc), lambda i: (i, 0)),
        out_shape=jax.ShapeDtypeStruct((E_PAD, oc), jnp.float32),
        compiler_params=_tc_params,
    )(rho, th, ph, zsrc, w0r, cw, seg)


def _make_post_body(final):
    def body(*refs):
        if final:
            a_r, z_r, h_r, b_r, x_r, o_r = refs
        else:
            a_r, z_r, h_r, b_r, o_r = refs
        acc = (a_r[0] + a_r[1]
               + jnp.dot(z_r[...], h_r[...], precision=_HI,
                         preferred_element_type=jnp.float32)
               + b_r[...])
        xn = _selu(acc)
        if final:
            o_r[...] = x_r[...] + BETA * xn
        else:
            o_r[...] = xn
    return body


def _tc_post(agg2, z, h0t, bias, x_in=None):
    oc = agg2.shape[-1]
    args = (agg2, z, h0t, bias) + (() if x_in is None else (x_in,))
    return pl.pallas_call(
        _make_post_body(x_in is not None),
        out_shape=jax.ShapeDtypeStruct((N_PAD, oc), jnp.float32),
        compiler_params=_tc_params,
    )(*args)


def kernel(x, edge_index, pos, W0_1, b0_1, W2_1, b2_1, bias_1,
           W0_2, b0_2, W2_2, b2_2, bias_2, W0_3, b0_3, W2_3, b2_3, bias_3):
    f32 = jnp.float32
    src = edge_index[0].astype(jnp.int32)
    dst = edge_index[1].astype(jnp.int32)
    pad = E_PAD - N_EDGES
    src_p = jnp.concatenate([src, jnp.zeros((pad,), jnp.int32)])
    dst_p = jnp.concatenate([dst, jnp.full((pad,), N_NODES, jnp.int32)])
    idx_src = src_p.reshape(NW, NCHUNK, CHUNK)
    idx_dst = dst_p.reshape(NW, NCHUNK, CHUNK)

    pos16 = jnp.concatenate([pos, jnp.zeros((N_NODES, 13), f32)], axis=1)

    def bfr(v):
        return v.astype(jnp.bfloat16).astype(f32)

    def wz(W2, b2):
        ic = W2.shape[1]
        m = jnp.concatenate([bfr(W2).T, b2[:, None]], axis=1)
        return jnp.concatenate([m, jnp.zeros((ic, ZW - HIDDEN - 1), f32)], 1)

    def consts(W0, b0, oc):
        w = bfr(W0[3])
        c = jnp.arange(oc, dtype=f32)
        cw = (c[:, None] * w[None, :] + b0[None, :]).reshape(1, oc * HIDDEN)
        seg = jnp.repeat(jnp.eye(oc, dtype=f32), HIDDEN, axis=0)
        h0 = bfr(_selu(b0[None, :] + c[:, None] * w[None, :]))
        h0 = jnp.concatenate(
            [h0, jnp.ones((oc, 1), f32), jnp.zeros((oc, ZW - HIDDEN - 1), f32)],
            axis=1)
        return bfr(W0[:3]), cw, seg, h0.T

    zeros16 = jnp.zeros((N_PAD, GROWTH), f32)
    zeros32 = jnp.zeros((N_PAD, IN_CH), f32)
    x_pad = jnp.concatenate([x, jnp.zeros((N_PAD - N_NODES, IN_CH), f32)])

    pos_g = _sc_gather_pos(pos16, idx_src, idx_dst)
    rho, th, ph = _tc_feat(pos_g.reshape(6, E_PAD // 128, 128))
    rho = rho.reshape(E_PAD, 1)
    th = th.reshape(E_PAD, 1)
    ph = ph.reshape(E_PAD, 1)

    def layer(xcat, W0, b0, W2, b2, bias, oc, zeros, x_in=None):
        w0r, cw, seg, h0t = consts(W0, b0, oc)
        z = _tc_matmul(xcat, wz(W2, b2))
        zsrc = _sc_gather_z(z, idx_src)
        msg = _tc_msg(rho, th, ph, zsrc, w0r, cw, seg, oc)
        agg2 = (_sc_scatter_16 if oc == GROWTH else _sc_scatter_32)(
            msg, idx_dst, zeros)
        return _tc_post(agg2, z, h0t, bias, x_in)

    x1 = layer(x_pad, W0_1, b0_1, W2_1, b2_1, bias_1, GROWTH, zeros16)
    x2 = layer(jnp.concatenate([x_pad, x1], axis=1),
               W0_2, b0_2, W2_2, b2_2, bias_2, GROWTH, zeros16)
    out = layer(jnp.concatenate([x_pad, x1, x2], axis=1),
                W0_3, b0_3, W2_3, b2_3, bias_3, IN_CH, zeros32, x_in=x_pad)
    return out[:N_NODES]


# speedup vs baseline: 1.3722x; 1.3722x over previous
import functools

import jax
import jax.numpy as jnp
from jax import lax
from jax.experimental import pallas as pl
from jax.experimental.pallas import tpu as pltpu
from jax.experimental.pallas import tpu_sc as plsc

N_NODES = 10000
N_EDGES = 80000
IN_CH = 32
GROWTH = 16
HIDDEN = 32
BETA = 0.2

NC = 2
NS = 16
NW = NC * NS
CHUNK = 128
EPW = 2560
NCHUNK = EPW // CHUNK
E_PAD = NW * EPW
N_PAD = 10240
ZW = 48

SELU_L = 1.0507009873554805
SELU_A = 1.6732632423543772

_mesh = plsc.VectorSubcoreMesh(core_axis_name="c", subcore_axis_name="s")
_sc_params = pltpu.CompilerParams(use_tc_tiling_on_sc=False,
                                  needs_layout_passes=False)


def _selu(x):
    return SELU_L * jnp.where(x > 0, x, SELU_A * (jnp.exp(x) - 1.0))


@functools.partial(
    pl.kernel,
    out_type=jax.ShapeDtypeStruct((6, E_PAD), jnp.float32),
    mesh=_mesh,
    compiler_params=_sc_params,
    scratch_types=[
        pltpu.VMEM((NCHUNK, CHUNK), jnp.int32),
        pltpu.VMEM((NCHUNK, CHUNK), jnp.int32),
        pltpu.VMEM((EPW, 16), jnp.float32),
        pltpu.VMEM((EPW, 16), jnp.float32),
        pltpu.VMEM((6, EPW), jnp.float32),
        pltpu.SemaphoreType.DMA,
    ],
)
def _sc_gather_pos(pos16, idx_s_hbm, idx_d_hbm, out, idx_s, idx_d,
                   rows_s, rows_d, cols, sem):
    wid = lax.axis_index("s") * NC + lax.axis_index("c")
    pltpu.sync_copy(idx_s_hbm.at[wid], idx_s)
    pltpu.sync_copy(idx_d_hbm.at[wid], idx_d)

    @pl.loop(0, NCHUNK)
    def _fire(j):
        pltpu.async_copy(pos16.at[idx_s.at[j]],
                         rows_s.at[pl.ds(j * CHUNK, CHUNK)], sem)
        pltpu.async_copy(pos16.at[idx_d.at[j]],
                         rows_d.at[pl.ds(j * CHUNK, CHUNK)], sem)

    @pl.loop(0, 2 * NCHUNK)
    def _drain(j):
        pltpu.make_async_copy(pos16.at[idx_s.at[0]],
                              rows_s.at[pl.ds(0, CHUNK)], sem).wait()

    lanes = lax.iota(jnp.int32, 16)

    @pl.loop(0, EPW // 16)
    def _tr(p):
        e_idx = lanes + p * 16
        for t in range(6):
            buf = rows_s if t < 3 else rows_d
            cidx = jnp.full((16,), t % 3, jnp.int32)
            cols[t, pl.ds(p * 16, 16)] = plsc.load_gather(buf, [e_idx, cidx])

    @pl.loop(0, 6)
    def _out(t):
        pltpu.sync_copy(cols.at[t], out.at[t].at[pl.ds(wid * EPW, EPW)])


@functools.partial(
    pl.kernel,
    out_type=jax.ShapeDtypeStruct((E_PAD, 128), jnp.float32),
    mesh=_mesh,
    compiler_params=_sc_params,
    scratch_types=[
        pltpu.VMEM((NCHUNK, CHUNK), jnp.int32),
        pltpu.VMEM((EPW, ZW), jnp.float32),
        pltpu.SemaphoreType.DMA,
    ],
)
def _sc_gather_z(table, idx_hbm, out, idx_v, rows, sem):
    wid = lax.axis_index("s") * NC + lax.axis_index("c")
    pltpu.sync_copy(idx_hbm.at[wid], idx_v)

    @pl.loop(0, NCHUNK)
    def _fire(j):
        pltpu.async_copy(table.at[idx_v.at[j]],
                         rows.at[pl.ds(j * CHUNK, CHUNK)], sem)

    @pl.loop(0, NCHUNK)
    def _drain(j):
        pltpu.make_async_copy(table.at[idx_v.at[0]],
                              rows.at[pl.ds(0, CHUNK)], sem).wait()

    pltpu.sync_copy(rows, out.at[pl.ds(wid * EPW, EPW), pl.ds(0, ZW)])


def _make_sc_scatter(oc):
    rpt = N_PAD // NS

    @functools.partial(
        pl.kernel,
        out_type=jax.ShapeDtypeStruct((NC, N_PAD, oc), jnp.float32),
        mesh=_mesh,
        compiler_params=_sc_params,
        scratch_types=[
            pltpu.VMEM((NCHUNK, CHUNK), jnp.int32),
            pltpu.VMEM((EPW, oc), jnp.float32),
            pltpu.VMEM_SHARED((N_PAD, oc), jnp.float32),
            pltpu.SemaphoreType.DMA,
        ],
    )
    def _sc_scatter(msg_hbm, idx_hbm, zeros_hbm, out, idx_v, msg_v, acc, sem):
        cid = lax.axis_index("c")
        sid = lax.axis_index("s")
        wid = sid * NC + cid
        pltpu.sync_copy(zeros_hbm.at[pl.ds(sid * rpt, rpt)],
                        acc.at[pl.ds(sid * rpt, rpt)])
        pltpu.sync_copy(idx_hbm.at[wid], idx_v)
        pltpu.sync_copy(msg_hbm.at[pl.ds(wid * EPW, EPW)], msg_v)
        plsc.subcore_barrier()

        @pl.loop(0, NCHUNK)
        def _scat(j):
            pltpu.sync_copy(msg_v.at[pl.ds(j * CHUNK, CHUNK)],
                            acc.at[idx_v.at[j]], add=True)

        plsc.subcore_barrier()
        pltpu.sync_copy(acc.at[pl.ds(sid * rpt, rpt)],
                        out.at[cid].at[pl.ds(sid * rpt, rpt)])

    return _sc_scatter


_sc_scatter_16 = _make_sc_scatter(GROWTH)
_sc_scatter_32 = _make_sc_scatter(IN_CH)


_HI = lax.Precision.HIGHEST
_tc_params = pltpu.CompilerParams(vmem_limit_bytes=60000 * 1024)


def _mm_body(x_r, w_r, o_r):
    o_r[...] = jnp.dot(x_r[...], w_r[...], precision=_HI,
                       preferred_element_type=jnp.float32)


def _tc_matmul(x, w):
    m, k = x.shape
    _, n = w.shape
    return pl.pallas_call(
        _mm_body,
        out_shape=jax.ShapeDtypeStruct((m, n), jnp.float32),
        compiler_params=_tc_params,
    )(x, w)


_PI = 3.141592653589793


def _atan(t):
    s = jnp.sign(t)
    a = jnp.abs(t)
    inv = a > 1.0
    a1 = jnp.where(inv, 1.0 / jnp.maximum(a, 1e-30), a)
    big = a1 > 0.4142135623730951
    a2 = jnp.where(big, (a1 - 1.0) / (a1 + 1.0), a1)
    z = a2 * a2
    y = ((8.05374449538e-2 * z - 1.38776856032e-1) * z
         + 1.99777106478e-1) * z - 3.33329491539e-1
    y = y * z * a2 + a2
    y = jnp.where(big, y + 0.7853981633974483, y)
    y = jnp.where(inv, 1.5707963267948966 - y, y)
    return s * y


def _atan2(y, x):
    ax = jnp.abs(x)
    ay = jnp.abs(y)
    t = ay / jnp.where(ax > 0, ax, 1.0)
    r = _atan(t)
    r = jnp.where(x < 0, _PI - r, r)
    r = jnp.where(jnp.maximum(ax, ay) == 0.0, 0.0, r)
    return jnp.where(y < 0, -r, r)


def _feat_body(p_r, orho, oth, oph):
    rx = p_r[3] - p_r[0]
    ry = p_r[4] - p_r[1]
    rz = p_r[5] - p_r[2]
    rho = jnp.sqrt(rx * rx + ry * ry + rz * rz)
    theta = _atan2(ry, rx)
    safe = jnp.where(rho > 0, rho, 1.0)
    u = jnp.clip(rz / safe, -1.0, 1.0)
    phi = _atan2(u, jnp.sqrt(jnp.maximum(1.0 - u * u, 0.0)))
    m = rho > 0
    orho[...] = rho
    oth[...] = jnp.where(m, theta, 0.0) * (1.0 / _PI)
    oph[...] = jnp.where(m, phi, 0.0) * (1.0 / _PI)


def _tc_feat(pos_g):
    rows = E_PAD // 128
    shp = jax.ShapeDtypeStruct((rows, 128), jnp.float32)
    return pl.pallas_call(
        _feat_body,
        out_shape=(shp, shp, shp),
        compiler_params=_tc_params,
    )(pos_g)


def _make_msg_body(oc, nblk):
    def body(rho_r, th_r, ph_r, z_r, w0_r, cw_r, seg_r, o_r):
        def bf(v):
            return v.astype(jnp.bfloat16).astype(jnp.float32)
        g0 = (bf(rho_r[...]) * w0_r[0:1, :] + bf(th_r[...]) * w0_r[1:2, :]
              + bf(ph_r[...]) * w0_r[2:3, :])
        gfull = jnp.concatenate([g0] * oc, axis=1) + cw_r[...]
        s_elu = bf(_selu(gfull))
        z = z_r[...]
        zt = jnp.concatenate([z[:, :HIDDEN]] * oc, axis=1)
        a = s_elu * zt
        a_hi = a.astype(jnp.bfloat16)
        a_lo = (a - a_hi.astype(jnp.float32)).astype(jnp.bfloat16)
        msg = (jnp.dot(a_hi, seg_r[...], preferred_element_type=jnp.float32)
               + jnp.dot(a_lo, seg_r[...], preferred_element_type=jnp.float32))
        o_r[...] = msg + z[:, HIDDEN:HIDDEN + 1]
    return body


def _tc_msg(rho, th, ph, zsrc, w0r, cw, seg, oc, blk=2048):
    nblk = E_PAD // blk
    k = oc * HIDDEN
    return pl.pallas_call(
        _make_msg_body(oc, nblk),
        grid=(nblk,),
        in_specs=[
            pl.BlockSpec((blk, 1), lambda i: (i, 0)),
            pl.BlockSpec((blk, 1), lambda i: (i, 0)),
            pl.BlockSpec((blk, 1), lambda i: (i, 0)),
            pl.BlockSpec((blk, 128), lambda i: (i, 0)),
            pl.BlockSpec((3, HIDDEN), lambda i: (0, 0)),
            pl.BlockSpec((1, k), lambda i: (0, 0)),
            pl.BlockSpec((k, oc), lambda i: (0, 0)),
        ],
        out_specs=pl.BlockSpec((blk, oc), lambda i: (i, 0)),
        out_shape=jax.ShapeDtypeStruct((E_PAD, oc), jnp.float32),
        compiler_params=_tc_params,
    )(rho, th, ph, zsrc, w0r, cw, seg)


def _make_post_body(final):
    def body(*refs):
        if final:
            a_r, z_r, h_r, b_r, x_r, o_r = refs
        else:
            a_r, z_r, h_r, b_r, o_r = refs
        acc = (a_r[0] + a_r[1]
               + jnp.dot(z_r[...], h_r[...], precision=_HI,
                         preferred_element_type=jnp.float32)
               + b_r[...])
        xn = _selu(acc)
        if final:
            o_r[...] = x_r[...] + BETA * xn
        else:
            o_r[...] = xn
    return body


def _tc_post(agg2, z, h0t, bias, x_in=None):
    oc = agg2.shape[-1]
    args = (agg2, z, h0t, bias) + (() if x_in is None else (x_in,))
    return pl.pallas_call(
        _make_post_body(x_in is not None),
        out_shape=jax.ShapeDtypeStruct((N_PAD, oc), jnp.float32),
        compiler_params=_tc_params,
    )(*args)


def kernel(x, edge_index, pos, W0_1, b0_1, W2_1, b2_1, bias_1,
           W0_2, b0_2, W2_2, b2_2, bias_2, W0_3, b0_3, W2_3, b2_3, bias_3):
    f32 = jnp.float32
    src = edge_index[0].astype(jnp.int32)
    dst = edge_index[1].astype(jnp.int32)
    pad = E_PAD - N_EDGES
    src_p = jnp.concatenate([src, jnp.zeros((pad,), jnp.int32)])
    dst_p = jnp.concatenate([dst, jnp.full((pad,), N_NODES, jnp.int32)])
    idx_src = src_p.reshape(NW, NCHUNK, CHUNK)
    idx_dst = dst_p.reshape(NW, NCHUNK, CHUNK)

    pos16 = jnp.concatenate([pos, jnp.zeros((N_NODES, 13), f32)], axis=1)

    def bfr(v):
        return v.astype(jnp.bfloat16).astype(f32)

    def wz(W2, b2):
        ic = W2.shape[1]
        m = jnp.concatenate([bfr(W2).T, b2[:, None]], axis=1)
        return jnp.concatenate([m, jnp.zeros((ic, ZW - HIDDEN - 1), f32)], 1)

    def consts(W0, b0, oc):
        w = bfr(W0[3])
        c = jnp.arange(oc, dtype=f32)
        cw = (c[:, None] * w[None, :] + b0[None, :]).reshape(1, oc * HIDDEN)
        seg = jnp.repeat(jnp.eye(oc, dtype=jnp.bfloat16), HIDDEN, axis=0)
        h0 = bfr(_selu(b0[None, :] + c[:, None] * w[None, :]))
        h0 = jnp.concatenate(
            [h0, jnp.ones((oc, 1), f32), jnp.zeros((oc, ZW - HIDDEN - 1), f32)],
            axis=1)
        return bfr(W0[:3]), cw, seg, h0.T

    zeros16 = jnp.zeros((N_PAD, GROWTH), f32)
    zeros32 = jnp.zeros((N_PAD, IN_CH), f32)
    x_pad = jnp.concatenate([x, jnp.zeros((N_PAD - N_NODES, IN_CH), f32)])

    pos_g = _sc_gather_pos(pos16, idx_src, idx_dst)
    rho, th, ph = _tc_feat(pos_g.reshape(6, E_PAD // 128, 128))
    rho = rho.reshape(E_PAD, 1)
    th = th.reshape(E_PAD, 1)
    ph = ph.reshape(E_PAD, 1)

    def layer(xcat, W0, b0, W2, b2, bias, oc, zeros, x_in=None):
        w0r, cw, seg, h0t = consts(W0, b0, oc)
        z = _tc_matmul(xcat, wz(W2, b2))
        zsrc = _sc_gather_z(z, idx_src)
        msg = _tc_msg(rho, th, ph, zsrc, w0r, cw, seg, oc)
        agg2 = (_sc_scatter_16 if oc == GROWTH else _sc_scatter_32)(
            msg, idx_dst, zeros)
        return _tc_post(agg2, z, h0t, bias, x_in)

    x1 = layer(x_pad, W0_1, b0_1, W2_1, b2_1, bias_1, GROWTH, zeros16)
    x2 = layer(jnp.concatenate([x_pad, x1], axis=1),
               W0_2, b0_2, W2_2, b2_2, bias_2, GROWTH, zeros16)
    out = layer(jnp.concatenate([x_pad, x1, x2], axis=1),
                W0_3, b0_3, W2_3, b2_3, bias_3, IN_CH, zeros32, x_in=x_pad)
    return out[:N_NODES]


# speedup vs baseline: 1.3904x; 1.0133x over previous
import functools

import jax
import jax.numpy as jnp
from jax import lax
from jax.experimental import pallas as pl
from jax.experimental.pallas import tpu as pltpu
from jax.experimental.pallas import tpu_sc as plsc

N_NODES = 10000
N_EDGES = 80000
IN_CH = 32
GROWTH = 16
HIDDEN = 32
BETA = 0.2

NC = 2
NS = 16
NW = NC * NS
CHUNK = 128
EPW = 2560
NCHUNK = EPW // CHUNK
E_PAD = NW * EPW
N_PAD = 10240
ZW = 48

SELU_L = 1.0507009873554805
SELU_A = 1.6732632423543772

_mesh = plsc.VectorSubcoreMesh(core_axis_name="c", subcore_axis_name="s")
_sc_params = pltpu.CompilerParams(use_tc_tiling_on_sc=False,
                                  needs_layout_passes=False)


def _selu(x):
    return SELU_L * jnp.where(x > 0, x, SELU_A * (jnp.exp(x) - 1.0))


@functools.partial(
    pl.kernel,
    out_type=jax.ShapeDtypeStruct((6, E_PAD), jnp.float32),
    mesh=_mesh,
    compiler_params=_sc_params,
    scratch_types=[
        pltpu.VMEM((NCHUNK, CHUNK), jnp.int32),
        pltpu.VMEM((NCHUNK, CHUNK), jnp.int32),
        pltpu.VMEM((EPW, 16), jnp.float32),
        pltpu.VMEM((EPW, 16), jnp.float32),
        pltpu.VMEM((6, EPW), jnp.float32),
        pltpu.SemaphoreType.DMA,
    ],
)
def _sc_gather_pos(pos16, idx_s_hbm, idx_d_hbm, out, idx_s, idx_d,
                   rows_s, rows_d, cols, sem):
    wid = lax.axis_index("s") * NC + lax.axis_index("c")
    pltpu.sync_copy(idx_s_hbm.at[wid], idx_s)
    pltpu.sync_copy(idx_d_hbm.at[wid], idx_d)

    @pl.loop(0, NCHUNK)
    def _fire(j):
        pltpu.async_copy(pos16.at[idx_s.at[j]],
                         rows_s.at[pl.ds(j * CHUNK, CHUNK)], sem)
        pltpu.async_copy(pos16.at[idx_d.at[j]],
                         rows_d.at[pl.ds(j * CHUNK, CHUNK)], sem)

    @pl.loop(0, 2 * NCHUNK)
    def _drain(j):
        pltpu.make_async_copy(pos16.at[idx_s.at[0]],
                              rows_s.at[pl.ds(0, CHUNK)], sem).wait()

    lanes = lax.iota(jnp.int32, 16)

    @pl.loop(0, EPW // 16)
    def _tr(p):
        e_idx = lanes + p * 16
        for t in range(6):
            buf = rows_s if t < 3 else rows_d
            cidx = jnp.full((16,), t % 3, jnp.int32)
            cols[t, pl.ds(p * 16, 16)] = plsc.load_gather(buf, [e_idx, cidx])

    @pl.loop(0, 6)
    def _out(t):
        pltpu.sync_copy(cols.at[t], out.at[t].at[pl.ds(wid * EPW, EPW)])


@functools.partial(
    pl.kernel,
    out_type=jax.ShapeDtypeStruct((E_PAD, 128), jnp.float32),
    mesh=_mesh,
    compiler_params=_sc_params,
    scratch_types=[
        pltpu.VMEM((NCHUNK, CHUNK), jnp.int32),
        pltpu.VMEM((EPW, ZW), jnp.float32),
        pltpu.SemaphoreType.DMA,
    ],
)
def _sc_gather_z(table, idx_hbm, out, idx_v, rows, sem):
    wid = lax.axis_index("s") * NC + lax.axis_index("c")
    pltpu.sync_copy(idx_hbm.at[wid], idx_v)

    @pl.loop(0, NCHUNK)
    def _fire(j):
        pltpu.async_copy(table.at[idx_v.at[j]],
                         rows.at[pl.ds(j * CHUNK, CHUNK)], sem)

    @pl.loop(0, NCHUNK)
    def _drain(j):
        pltpu.make_async_copy(table.at[idx_v.at[0]],
                              rows.at[pl.ds(0, CHUNK)], sem).wait()

    pltpu.sync_copy(rows, out.at[pl.ds(wid * EPW, EPW), pl.ds(0, ZW)])


def _make_sc_scatter(oc):
    rpt = N_PAD // NS

    @functools.partial(
        pl.kernel,
        out_type=jax.ShapeDtypeStruct((NC, N_PAD, oc), jnp.float32),
        mesh=_mesh,
        compiler_params=_sc_params,
        scratch_types=[
            pltpu.VMEM((NCHUNK, CHUNK), jnp.int32),
            pltpu.VMEM((EPW, oc), jnp.float32),
            pltpu.VMEM_SHARED((N_PAD, oc), jnp.float32),
            pltpu.SemaphoreType.DMA,
        ],
    )
    def _sc_scatter(msg_hbm, idx_hbm, zeros_hbm, out, idx_v, msg_v, acc, sem):
        cid = lax.axis_index("c")
        sid = lax.axis_index("s")
        wid = sid * NC + cid
        pltpu.sync_copy(zeros_hbm.at[pl.ds(sid * rpt, rpt)],
                        acc.at[pl.ds(sid * rpt, rpt)])
        pltpu.sync_copy(idx_hbm.at[wid], idx_v)
        pltpu.sync_copy(msg_hbm.at[pl.ds(wid * EPW, EPW), pl.ds(0, oc)], msg_v)
        plsc.subcore_barrier()

        @pl.loop(0, NCHUNK)
        def _scat(j):
            pltpu.sync_copy(msg_v.at[pl.ds(j * CHUNK, CHUNK)],
                            acc.at[idx_v.at[j]], add=True)

        plsc.subcore_barrier()
        pltpu.sync_copy(acc.at[pl.ds(sid * rpt, rpt)],
                        out.at[cid].at[pl.ds(sid * rpt, rpt)])

    return _sc_scatter


_sc_scatter_16 = _make_sc_scatter(GROWTH)
_sc_scatter_32 = _make_sc_scatter(IN_CH)


_HI = lax.Precision.HIGHEST
_tc_params = pltpu.CompilerParams(vmem_limit_bytes=60000 * 1024)


def _mm_body(x_r, w_r, o_r):
    o_r[...] = jnp.dot(x_r[...], w_r[...], precision=_HI,
                       preferred_element_type=jnp.float32)


def _tc_matmul(x, w):
    m, k = x.shape
    _, n = w.shape
    return pl.pallas_call(
        _mm_body,
        out_shape=jax.ShapeDtypeStruct((m, n), jnp.float32),
        compiler_params=_tc_params,
    )(x, w)


_PI = 3.141592653589793


def _atan(t):
    s = jnp.sign(t)
    a = jnp.abs(t)
    inv = a > 1.0
    a1 = jnp.where(inv, 1.0 / jnp.maximum(a, 1e-30), a)
    big = a1 > 0.4142135623730951
    a2 = jnp.where(big, (a1 - 1.0) / (a1 + 1.0), a1)
    z = a2 * a2
    y = ((8.05374449538e-2 * z - 1.38776856032e-1) * z
         + 1.99777106478e-1) * z - 3.33329491539e-1
    y = y * z * a2 + a2
    y = jnp.where(big, y + 0.7853981633974483, y)
    y = jnp.where(inv, 1.5707963267948966 - y, y)
    return s * y


def _atan2(y, x):
    ax = jnp.abs(x)
    ay = jnp.abs(y)
    t = ay / jnp.where(ax > 0, ax, 1.0)
    r = _atan(t)
    r = jnp.where(x < 0, _PI - r, r)
    r = jnp.where(jnp.maximum(ax, ay) == 0.0, 0.0, r)
    return jnp.where(y < 0, -r, r)


def _feat_body(p_r, orho, oth, oph):
    rx = p_r[3] - p_r[0]
    ry = p_r[4] - p_r[1]
    rz = p_r[5] - p_r[2]
    rho = jnp.sqrt(rx * rx + ry * ry + rz * rz)
    theta = _atan2(ry, rx)
    safe = jnp.where(rho > 0, rho, 1.0)
    u = jnp.clip(rz / safe, -1.0, 1.0)
    phi = _atan2(u, jnp.sqrt(jnp.maximum(1.0 - u * u, 0.0)))
    m = rho > 0
    def bf(v):
        return v.astype(jnp.bfloat16).astype(jnp.float32)
    orho[...] = bf(rho)
    oth[...] = bf(jnp.where(m, theta, 0.0) * (1.0 / _PI))
    oph[...] = bf(jnp.where(m, phi, 0.0) * (1.0 / _PI))


def _tc_feat(pos_g):
    rows = E_PAD // 128
    shp = jax.ShapeDtypeStruct((rows, 128), jnp.float32)
    return pl.pallas_call(
        _feat_body,
        out_shape=(shp, shp, shp),
        compiler_params=_tc_params,
    )(pos_g)


def _make_msg_body(oc, nblk):
    def body(rho_r, th_r, ph_r, z_r, w0_r, cw_r, seg_r, o_r):
        def bf(v):
            return v.astype(jnp.bfloat16).astype(jnp.float32)
        g0 = (rho_r[...] * w0_r[0:1, :] + th_r[...] * w0_r[1:2, :]
              + ph_r[...] * w0_r[2:3, :])
        gfull = jnp.concatenate([g0] * oc, axis=1) + cw_r[...]
        s_elu = bf(_selu(gfull))
        z = z_r[...]
        zt = jnp.concatenate([z[:, :HIDDEN]] * oc, axis=1)
        a = s_elu * zt
        a_hi = a.astype(jnp.bfloat16)
        a_lo = (a - a_hi.astype(jnp.float32)).astype(jnp.bfloat16)
        msg = (jnp.dot(a_hi, seg_r[...], preferred_element_type=jnp.float32)
               + jnp.dot(a_lo, seg_r[...], preferred_element_type=jnp.float32))
        o_r[:, pl.ds(0, oc)] = msg + z[:, HIDDEN:HIDDEN + 1]
    return body


def _tc_msg(rho, th, ph, zsrc, w0r, cw, seg, oc, blk=2048):
    nblk = E_PAD // blk
    k = oc * HIDDEN
    return pl.pallas_call(
        _make_msg_body(oc, nblk),
        grid=(nblk,),
        in_specs=[
            pl.BlockSpec((blk, 1), lambda i: (i, 0)),
            pl.BlockSpec((blk, 1), lambda i: (i, 0)),
            pl.BlockSpec((blk, 1), lambda i: (i, 0)),
            pl.BlockSpec((blk, 128), lambda i: (i, 0)),
            pl.BlockSpec((3, HIDDEN), lambda i: (0, 0)),
            pl.BlockSpec((1, k), lambda i: (0, 0)),
            pl.BlockSpec((k, oc), lambda i: (0, 0)),
        ],
        out_specs=pl.BlockSpec((blk, 128), lambda i: (i, 0)),
        out_shape=jax.ShapeDtypeStruct((E_PAD, 128), jnp.float32),
        compiler_params=_tc_params,
    )(rho, th, ph, zsrc, w0r, cw, seg)


def _make_post_body(final):
    def body(*refs):
        if final:
            a_r, z_r, h_r, b_r, x_r, o_r = refs
        else:
            a_r, z_r, h_r, b_r, o_r = refs
        acc = (a_r[0] + a_r[1]
               + jnp.dot(z_r[...], h_r[...], precision=_HI,
                         preferred_element_type=jnp.float32)
               + b_r[...])
        xn = _selu(acc)
        if final:
            o_r[...] = x_r[...] + BETA * xn
        else:
            o_r[...] = xn
    return body


def _tc_post(agg2, z, h0t, bias, x_in=None):
    oc = agg2.shape[-1]
    args = (agg2, z, h0t, bias) + (() if x_in is None else (x_in,))
    return pl.pallas_call(
        _make_post_body(x_in is not None),
        out_shape=jax.ShapeDtypeStruct((N_PAD, oc), jnp.float32),
        compiler_params=_tc_params,
    )(*args)


def kernel(x, edge_index, pos, W0_1, b0_1, W2_1, b2_1, bias_1,
           W0_2, b0_2, W2_2, b2_2, bias_2, W0_3, b0_3, W2_3, b2_3, bias_3):
    f32 = jnp.float32
    src = edge_index[0].astype(jnp.int32)
    dst = edge_index[1].astype(jnp.int32)
    pad = E_PAD - N_EDGES
    src_p = jnp.concatenate([src, jnp.zeros((pad,), jnp.int32)])
    dst_p = jnp.concatenate([dst, jnp.full((pad,), N_NODES, jnp.int32)])
    idx_src = src_p.reshape(NW, NCHUNK, CHUNK)
    idx_dst = dst_p.reshape(NW, NCHUNK, CHUNK)

    pos16 = jnp.concatenate([pos, jnp.zeros((N_NODES, 13), f32)], axis=1)

    def bfr(v):
        return v.astype(jnp.bfloat16).astype(f32)

    def wz(W2, b2):
        ic = W2.shape[1]
        m = jnp.concatenate([bfr(W2).T, b2[:, None]], axis=1)
        return jnp.concatenate([m, jnp.zeros((ic, ZW - HIDDEN - 1), f32)], 1)

    def consts(W0, b0, oc):
        w = bfr(W0[3])
        c = jnp.arange(oc, dtype=f32)
        cw = (c[:, None] * w[None, :] + b0[None, :]).reshape(1, oc * HIDDEN)
        seg = jnp.repeat(jnp.eye(oc, dtype=jnp.bfloat16), HIDDEN, axis=0)
        h0 = bfr(_selu(b0[None, :] + c[:, None] * w[None, :]))
        h0 = jnp.concatenate(
            [h0, jnp.ones((oc, 1), f32), jnp.zeros((oc, ZW - HIDDEN - 1), f32)],
            axis=1)
        return bfr(W0[:3]), cw, seg, h0.T

    zeros16 = jnp.zeros((N_PAD, GROWTH), f32)
    zeros32 = jnp.zeros((N_PAD, IN_CH), f32)
    x_pad = jnp.concatenate([x, jnp.zeros((N_PAD - N_NODES, IN_CH), f32)])

    pos_g = _sc_gather_pos(pos16, idx_src, idx_dst)
    rho, th, ph = _tc_feat(pos_g.reshape(6, E_PAD // 128, 128))
    rho = rho.reshape(E_PAD, 1)
    th = th.reshape(E_PAD, 1)
    ph = ph.reshape(E_PAD, 1)

    def layer(xcat, W0, b0, W2, b2, bias, oc, zeros, x_in=None):
        w0r, cw, seg, h0t = consts(W0, b0, oc)
        z = _tc_matmul(xcat, wz(W2, b2))
        zsrc = _sc_gather_z(z, idx_src)
        msg = _tc_msg(rho, th, ph, zsrc, w0r, cw, seg, oc)
        agg2 = (_sc_scatter_16 if oc == GROWTH else _sc_scatter_32)(
            msg, idx_dst, zeros)
        return _tc_post(agg2, z, h0t, bias, x_in)

    x1 = layer(x_pad, W0_1, b0_1, W2_1, b2_1, bias_1, GROWTH, zeros16)
    x2 = layer(jnp.concatenate([x_pad, x1], axis=1),
               W0_2, b0_2, W2_2, b2_2, bias_2, GROWTH, zeros16)
    out = layer(jnp.concatenate([x_pad, x1, x2], axis=1),
                W0_3, b0_3, W2_3, b2_3, bias_3, IN_CH, zeros32, x_in=x_pad)
    return out[:N_NODES]
